# Initial kernel scaffold; baseline (speedup 1.0000x reference)
import functools

import jax
import jax.numpy as jnp
from jax import lax
from jax.experimental import pallas as pl
from jax.experimental.pallas import tpu as pltpu
from jax.experimental.pallas import tpu_sc as plsc

N_NODES = 10000
N_EDGES = 320000
H = 64
NUM_LAYERS = 3

NC = 2
NS = 16
NW = NC * NS
EPW = N_EDGES // NW
CH = 80
NCH = EPW // CH
ROWS_PER_TILE = N_NODES // NS

_f32 = jnp.float32

_sc_mesh = plsc.VectorSubcoreMesh(core_axis_name="c", subcore_axis_name="s")


@functools.partial(
    pl.kernel,
    out_type=(
        jax.ShapeDtypeStruct((N_EDGES, H), _f32),
        jax.ShapeDtypeStruct((N_EDGES, H), _f32),
    ),
    mesh=_sc_mesh,
    scratch_types=[
        pltpu.VMEM((CH,), jnp.int32),
        pltpu.VMEM((CH,), jnp.int32),
        pltpu.VMEM((CH, H), _f32),
        pltpu.VMEM((CH, H), _f32),
        pltpu.SemaphoreType.DMA,
        pltpu.SemaphoreType.DMA,
    ],
)
def _sc_gather(h_hbm, row_hbm, col_hbm, gr_hbm, gc_hbm,
               ridx, cidx, rbuf, cbuf, sem_r, sem_c):
    wid = lax.axis_index("s") * NC + lax.axis_index("c")
    base = wid * EPW

    def body(j, carry):
        off = base + j * CH
        pltpu.sync_copy(row_hbm.at[pl.ds(off, CH)], ridx)
        pltpu.sync_copy(col_hbm.at[pl.ds(off, CH)], cidx)
        cpr = pltpu.async_copy(h_hbm.at[ridx], rbuf, sem_r)
        cpc = pltpu.async_copy(h_hbm.at[cidx], cbuf, sem_c)
        cpr.wait()
        pltpu.sync_copy(rbuf, gr_hbm.at[pl.ds(off, CH)])
        cpc.wait()
        pltpu.sync_copy(cbuf, gc_hbm.at[pl.ds(off, CH)])
        return carry

    lax.fori_loop(0, NCH, body, 0)


@functools.partial(
    pl.kernel,
    out_type=jax.ShapeDtypeStruct((NC * N_NODES, H), _f32),
    mesh=_sc_mesh,
    scratch_types=[
        pltpu.VMEM((CH,), jnp.int32),
        pltpu.VMEM((CH, H), _f32),
        pltpu.VMEM_SHARED((N_NODES, H), _f32),
    ],
)
def _sc_scatter(enew_hbm, col_hbm, zeros_hbm, out_hbm, cidx, buf, acc):
    cid = lax.axis_index("c")
    sid = lax.axis_index("s")
    wid = sid * NC + cid
    r0 = sid * ROWS_PER_TILE
    pltpu.sync_copy(zeros_hbm.at[pl.ds(r0, ROWS_PER_TILE)],
                    acc.at[pl.ds(r0, ROWS_PER_TILE)])
    plsc.subcore_barrier()
    base = wid * EPW

    def body(j, carry):
        off = base + j * CH
        pltpu.sync_copy(col_hbm.at[pl.ds(off, CH)], cidx)
        pltpu.sync_copy(enew_hbm.at[pl.ds(off, CH)], buf)
        pltpu.sync_copy(buf, acc.at[cidx], add=True)
        return carry

    lax.fori_loop(0, NCH, body, 0)
    plsc.subcore_barrier()
    pltpu.sync_copy(acc.at[pl.ds(r0, ROWS_PER_TILE)],
                    out_hbm.at[pl.ds(cid * N_NODES + r0, ROWS_PER_TILE)])


def _ln(t, g, b):
    mu = jnp.mean(t, axis=-1, keepdims=True)
    d = t - mu
    var = jnp.mean(d * d, axis=-1, keepdims=True)
    return d * lax.rsqrt(var + 1e-5) * g + b


def _dot(a, b):
    return jnp.dot(a, b, preferred_element_type=_f32)


def _init_body(x, W, b, hout):
    hout[...] = _dot(x[...], W[...]) + b[...]


def _edge0_body(gr, gc, ea, edW, edb, W1r, W1c, W1e, b1, g1, be1,
                W2, b2, g2, be2, enew, enext):
    ev = _dot(ea[...], edW[...]) + edb[...]
    t = (_dot(gr[...], W1r[...]) + _dot(gc[...], W1c[...])
         + _dot(ev, W1e[...]) + b1[...])
    t = jnp.maximum(_ln(t, g1[...], be1[...]), 0.0)
    u = _ln(_dot(t, W2[...]) + b2[...], g2[...], be2[...])
    enew[...] = u
    enext[...] = ev + u


def _edge_body(gr, gc, e, W1r, W1c, W1e, b1, g1, be1,
               W2, b2, g2, be2, enew, enext):
    ev = e[...]
    t = (_dot(gr[...], W1r[...]) + _dot(gc[...], W1c[...])
         + _dot(ev, W1e[...]) + b1[...])
    t = jnp.maximum(_ln(t, g1[...], be1[...]), 0.0)
    u = _ln(_dot(t, W2[...]) + b2[...], g2[...], be2[...])
    enew[...] = u
    enext[...] = ev + u


def _node_body(h, parts, W1h, W1a, b1, g1, be1, W2, b2, g2, be2, hout):
    hv = h[...]
    a = parts[0] + parts[1]
    t = _dot(hv, W1h[...]) + _dot(a, W1a[...]) + b1[...]
    t = jnp.maximum(_ln(t, g1[...], be1[...]), 0.0)
    u = _ln(_dot(t, W2[...]) + b2[...], g2[...], be2[...])
    hout[...] = hv + u


BE = 2000
BN = 2000


def _espec():
    return pl.BlockSpec((BE, H), lambda i: (i, 0))


def _wspec(shape):
    return pl.BlockSpec(shape, lambda i, _s=len(shape): (0,) * _s)


def _tc_init(x, W, b):
    return pl.pallas_call(
        _init_body,
        grid=(N_NODES // BN,),
        in_specs=[pl.BlockSpec((BN, x.shape[1]), lambda i: (i, 0)),
                  _wspec(W.shape), _wspec(b.shape)],
        out_specs=pl.BlockSpec((BN, H), lambda i: (i, 0)),
        out_shape=jax.ShapeDtypeStruct((N_NODES, H), _f32),
    )(x, W, b)


def _tc_edge(body, arrays, weights):
    aspecs = [pl.BlockSpec((BE, a.shape[1]), lambda i: (i, 0)) for a in arrays]
    wspecs = [_wspec(w.shape) for w in weights]
    return pl.pallas_call(
        body,
        grid=(N_EDGES // BE,),
        in_specs=aspecs + wspecs,
        out_specs=(_espec(), _espec()),
        out_shape=(jax.ShapeDtypeStruct((N_EDGES, H), _f32),
                   jax.ShapeDtypeStruct((N_EDGES, H), _f32)),
    )(*arrays, *weights)


def _tc_node(h, parts, weights):
    return pl.pallas_call(
        _node_body,
        grid=(N_NODES // BN,),
        in_specs=[pl.BlockSpec((BN, H), lambda i: (i, 0)),
                  pl.BlockSpec((NC, BN, H), lambda i: (0, i, 0))]
                 + [_wspec(w.shape) for w in weights],
        out_specs=pl.BlockSpec((BN, H), lambda i: (i, 0)),
        out_shape=jax.ShapeDtypeStruct((N_NODES, H), _f32),
    )(h, parts, *weights)


def kernel(x, edge_index, edge_attr, params):
    p = params
    row = edge_index[0]
    col = edge_index[1]
    zeros_nodes = jnp.zeros((N_NODES, H), _f32)

    def r1(v):
        return v.reshape(1, H)

    h = _tc_init(x, p['in_W'], r1(p['in_b']))
    e = None
    for l in range(NUM_LAYERS):
        pe = 'l%d_e_' % l
        pn = 'l%d_n_' % l
        W1 = p[pe + 'W1']
        W1r, W1c, W1e = W1[:H], W1[H:2 * H], W1[2 * H:]
        ew = [W1r, W1c, W1e, r1(p[pe + 'b1']), r1(p[pe + 'g1']),
              r1(p[pe + 'be1']), p[pe + 'W2'], r1(p[pe + 'b2']),
              r1(p[pe + 'g2']), r1(p[pe + 'be2'])]
        gr, gc = _sc_gather(h, row, col)
        if l == 0:
            e_new, e = _tc_edge(
                _edge0_body, [gr, gc, edge_attr],
                [p['ed_W'], r1(p['ed_b'])] + ew)
        else:
            e_new, e = _tc_edge(_edge_body, [gr, gc, e], ew)
        parts = _sc_scatter(e_new, col, zeros_nodes).reshape(NC, N_NODES, H)
        nW1 = p[pn + 'W1']
        nw = [nW1[:H], nW1[H:], r1(p[pn + 'b1']), r1(p[pn + 'g1']),
              r1(p[pn + 'be1']), p[pn + 'W2'], r1(p[pn + 'b2']),
              r1(p[pn + 'g2']), r1(p[pn + 'be2'])]
        h = _tc_node(h, parts, nw)
    return (h, e)


# speedup vs baseline: 1.8435x; 1.8435x over previous
import functools

import jax
import jax.numpy as jnp
from jax import lax
from jax.experimental import pallas as pl
from jax.experimental.pallas import tpu as pltpu
from jax.experimental.pallas import tpu_sc as plsc

N_NODES = 10000
N_EDGES = 320000
H = 64
NUM_LAYERS = 3

NC = 2
NS = 16
NW = NC * NS
EPW = N_EDGES // NW
CH = 80
NCH = EPW // CH
ROWS_PER_TILE = N_NODES // NS

_f32 = jnp.float32

_sc_mesh = plsc.VectorSubcoreMesh(core_axis_name="c", subcore_axis_name="s")


@functools.partial(
    pl.kernel,
    out_type=(
        jax.ShapeDtypeStruct((N_EDGES, H), _f32),
        jax.ShapeDtypeStruct((N_EDGES, H), _f32),
    ),
    mesh=_sc_mesh,
    scratch_types=[
        pltpu.VMEM((CH,), jnp.int32),
        pltpu.VMEM((CH,), jnp.int32),
        pltpu.VMEM((CH, H), _f32),
        pltpu.VMEM((CH, H), _f32),
        pltpu.SemaphoreType.DMA,
        pltpu.SemaphoreType.DMA,
    ],
    compiler_params=pltpu.CompilerParams(use_tc_tiling_on_sc=False),
)
def _sc_gather(h_hbm, row_hbm, col_hbm, gr_hbm, gc_hbm,
               ridx, cidx, rbuf, cbuf, sem_r, sem_c):
    wid = lax.axis_index("s") * NC + lax.axis_index("c")
    base = wid * EPW

    def body(j, carry):
        off = base + j * CH
        pltpu.sync_copy(row_hbm.at[pl.ds(off, CH)], ridx)
        pltpu.sync_copy(col_hbm.at[pl.ds(off, CH)], cidx)
        cpr = pltpu.async_copy(h_hbm.at[ridx], rbuf, sem_r)
        cpc = pltpu.async_copy(h_hbm.at[cidx], cbuf, sem_c)
        cpr.wait()
        pltpu.sync_copy(rbuf, gr_hbm.at[pl.ds(off, CH)])
        cpc.wait()
        pltpu.sync_copy(cbuf, gc_hbm.at[pl.ds(off, CH)])
        return carry

    lax.fori_loop(0, NCH, body, 0)


@functools.partial(
    pl.kernel,
    out_type=jax.ShapeDtypeStruct((NC * N_NODES, H), _f32),
    mesh=_sc_mesh,
    scratch_types=[
        pltpu.VMEM((CH,), jnp.int32),
        pltpu.VMEM((CH, H), _f32),
        pltpu.VMEM_SHARED((N_NODES, H), _f32),
    ],
    compiler_params=pltpu.CompilerParams(use_tc_tiling_on_sc=False),
)
def _sc_scatter(enew_hbm, col_hbm, zeros_hbm, out_hbm, cidx, buf, acc):
    cid = lax.axis_index("c")
    sid = lax.axis_index("s")
    wid = sid * NC + cid
    r0 = sid * ROWS_PER_TILE
    pltpu.sync_copy(zeros_hbm.at[pl.ds(r0, ROWS_PER_TILE)],
                    acc.at[pl.ds(r0, ROWS_PER_TILE)])
    plsc.subcore_barrier()
    base = wid * EPW

    def body(j, carry):
        off = base + j * CH
        pltpu.sync_copy(col_hbm.at[pl.ds(off, CH)], cidx)
        pltpu.sync_copy(enew_hbm.at[pl.ds(off, CH)], buf)
        pltpu.sync_copy(buf, acc.at[cidx], add=True)
        return carry

    lax.fori_loop(0, NCH, body, 0)
    plsc.subcore_barrier()
    pltpu.sync_copy(acc.at[pl.ds(r0, ROWS_PER_TILE)],
                    out_hbm.at[pl.ds(cid * N_NODES + r0, ROWS_PER_TILE)])


def _ln(t, g, b):
    mu = jnp.mean(t, axis=-1, keepdims=True)
    d = t - mu
    var = jnp.mean(d * d, axis=-1, keepdims=True)
    return d * lax.rsqrt(var + 1e-5) * g + b


def _dot(a, b):
    return jnp.dot(a, b, preferred_element_type=_f32)


def _init_body(x, W, b, hout):
    hout[...] = _dot(x[...], W[...]) + b[...]


def _edge0_body(gr, gc, ea, edW, edb, W1r, W1c, W1e, b1, g1, be1,
                W2, b2, g2, be2, enew, enext):
    ev = _dot(ea[...], edW[...]) + edb[...]
    t = (_dot(gr[...], W1r[...]) + _dot(gc[...], W1c[...])
         + _dot(ev, W1e[...]) + b1[...])
    t = jnp.maximum(_ln(t, g1[...], be1[...]), 0.0)
    u = _ln(_dot(t, W2[...]) + b2[...], g2[...], be2[...])
    enew[...] = u
    enext[...] = ev + u


def _edge_body(gr, gc, e, W1r, W1c, W1e, b1, g1, be1,
               W2, b2, g2, be2, enew, enext):
    ev = e[...]
    t = (_dot(gr[...], W1r[...]) + _dot(gc[...], W1c[...])
         + _dot(ev, W1e[...]) + b1[...])
    t = jnp.maximum(_ln(t, g1[...], be1[...]), 0.0)
    u = _ln(_dot(t, W2[...]) + b2[...], g2[...], be2[...])
    enew[...] = u
    enext[...] = ev + u


def _node_body(h, parts, W1h, W1a, b1, g1, be1, W2, b2, g2, be2, hout):
    hv = h[...]
    a = parts[0] + parts[1]
    t = _dot(hv, W1h[...]) + _dot(a, W1a[...]) + b1[...]
    t = jnp.maximum(_ln(t, g1[...], be1[...]), 0.0)
    u = _ln(_dot(t, W2[...]) + b2[...], g2[...], be2[...])
    hout[...] = hv + u


BE = 2000
BN = 2000


def _espec():
    return pl.BlockSpec((BE, H), lambda i: (i, 0))


def _wspec(shape):
    return pl.BlockSpec(shape, lambda i, _s=len(shape): (0,) * _s)


def _tc_init(x, W, b):
    return pl.pallas_call(
        _init_body,
        grid=(N_NODES // BN,),
        in_specs=[pl.BlockSpec((BN, x.shape[1]), lambda i: (i, 0)),
                  _wspec(W.shape), _wspec(b.shape)],
        out_specs=pl.BlockSpec((BN, H), lambda i: (i, 0)),
        out_shape=jax.ShapeDtypeStruct((N_NODES, H), _f32),
    )(x, W, b)


def _tc_edge(body, arrays, weights):
    aspecs = [pl.BlockSpec((BE, a.shape[1]), lambda i: (i, 0)) for a in arrays]
    wspecs = [_wspec(w.shape) for w in weights]
    return pl.pallas_call(
        body,
        grid=(N_EDGES // BE,),
        in_specs=aspecs + wspecs,
        out_specs=(_espec(), _espec()),
        out_shape=(jax.ShapeDtypeStruct((N_EDGES, H), _f32),
                   jax.ShapeDtypeStruct((N_EDGES, H), _f32)),
    )(*arrays, *weights)


def _tc_node(h, parts, weights):
    return pl.pallas_call(
        _node_body,
        grid=(N_NODES // BN,),
        in_specs=[pl.BlockSpec((BN, H), lambda i: (i, 0)),
                  pl.BlockSpec((NC, BN, H), lambda i: (0, i, 0))]
                 + [_wspec(w.shape) for w in weights],
        out_specs=pl.BlockSpec((BN, H), lambda i: (i, 0)),
        out_shape=jax.ShapeDtypeStruct((N_NODES, H), _f32),
    )(h, parts, *weights)


def kernel(x, edge_index, edge_attr, params):
    p = params
    row = edge_index[0]
    col = edge_index[1]
    zeros_nodes = jnp.zeros((N_NODES, H), _f32)

    def r1(v):
        return v.reshape(1, H)

    h = _tc_init(x, p['in_W'], r1(p['in_b']))
    e = None
    for l in range(NUM_LAYERS):
        pe = 'l%d_e_' % l
        pn = 'l%d_n_' % l
        W1 = p[pe + 'W1']
        W1r, W1c, W1e = W1[:H], W1[H:2 * H], W1[2 * H:]
        ew = [W1r, W1c, W1e, r1(p[pe + 'b1']), r1(p[pe + 'g1']),
              r1(p[pe + 'be1']), p[pe + 'W2'], r1(p[pe + 'b2']),
              r1(p[pe + 'g2']), r1(p[pe + 'be2'])]
        gr, gc = _sc_gather(h, row, col)
        if l == 0:
            e_new, e = _tc_edge(
                _edge0_body, [gr, gc, edge_attr],
                [p['ed_W'], r1(p['ed_b'])] + ew)
        else:
            e_new, e = _tc_edge(_edge_body, [gr, gc, e], ew)
        parts = _sc_scatter(e_new, col, zeros_nodes).reshape(NC, N_NODES, H)
        nW1 = p[pn + 'W1']
        nw = [nW1[:H], nW1[H:], r1(p[pn + 'b1']), r1(p[pn + 'g1']),
              r1(p[pn + 'be1']), p[pn + 'W2'], r1(p[pn + 'b2']),
              r1(p[pn + 'g2']), r1(p[pn + 'be2'])]
        h = _tc_node(h, parts, nw)
    return (h, e)


# speedup vs baseline: 3.6580x; 1.9843x over previous
import functools

import jax
import jax.numpy as jnp
from jax import lax
from jax.experimental import pallas as pl
from jax.experimental.pallas import tpu as pltpu
from jax.experimental.pallas import tpu_sc as plsc

N_NODES = 10000
N_EDGES = 320000
H = 64
H2 = 2 * H
NUM_LAYERS = 3

NC = 2
NS = 16
NW = NC * NS
EPW = N_EDGES // NW
CH = 80
NCH = EPW // CH
GRP = 5
NG = NCH // GRP
GE = GRP * CH
ROWS_PER_TILE = N_NODES // NS

_f32 = jnp.float32

_sc_mesh = plsc.VectorSubcoreMesh(core_axis_name="c", subcore_axis_name="s")
_sc_params = pltpu.CompilerParams(use_tc_tiling_on_sc=False)


@functools.partial(
    pl.kernel,
    out_type=jax.ShapeDtypeStruct((N_EDGES, H2), _f32),
    mesh=_sc_mesh,
    scratch_types=[
        pltpu.VMEM((NCH, CH), jnp.int32),
        pltpu.VMEM((NCH, CH), jnp.int32),
        pltpu.VMEM((GE, H), _f32),
        pltpu.VMEM((GE, H), _f32),
        pltpu.SemaphoreType.DMA,
    ],
    compiler_params=_sc_params,
)
def _sc_gather(h_hbm, row_hbm, col_hbm, gcat_hbm,
               idx_r, idx_c, rbuf, cbuf, semg):
    wid = lax.axis_index("s") * NC + lax.axis_index("c")
    base = wid * EPW
    pltpu.sync_copy(row_hbm.at[wid], idx_r)
    pltpu.sync_copy(col_hbm.at[wid], idx_c)

    @pl.loop(0, NG)
    def _(g):
        goff = base + g * GE
        cps = []
        for k in range(GRP):
            ck = g * GRP + k
            cps.append(pltpu.async_copy(
                h_hbm.at[idx_r.at[ck]], rbuf.at[pl.ds(k * CH, CH)], semg))
            cps.append(pltpu.async_copy(
                h_hbm.at[idx_c.at[ck]], cbuf.at[pl.ds(k * CH, CH)], semg))
        for cp in cps:
            cp.wait()
        pltpu.sync_copy(rbuf, gcat_hbm.at[pl.ds(goff, GE), pl.ds(0, H)])
        pltpu.sync_copy(cbuf, gcat_hbm.at[pl.ds(goff, GE), pl.ds(H, H)])


@functools.partial(
    pl.kernel,
    out_type=jax.ShapeDtypeStruct((NC * N_NODES, H), _f32),
    mesh=_sc_mesh,
    scratch_types=[
        pltpu.VMEM((NCH, CH), jnp.int32),
        pltpu.VMEM((GE, H), _f32),
        pltpu.VMEM_SHARED((N_NODES, H), _f32),
        pltpu.SemaphoreType.DMA,
    ],
    compiler_params=_sc_params,
)
def _sc_scatter(epair_hbm, col_hbm, zeros_hbm, out_hbm, idx_c, ebuf, acc, sem):
    cid = lax.axis_index("c")
    sid = lax.axis_index("s")
    wid = sid * NC + cid
    r0 = sid * ROWS_PER_TILE
    pltpu.sync_copy(zeros_hbm.at[pl.ds(r0, ROWS_PER_TILE)],
                    acc.at[pl.ds(r0, ROWS_PER_TILE)])
    pltpu.sync_copy(col_hbm.at[wid], idx_c)
    plsc.subcore_barrier()
    base = wid * EPW

    @pl.loop(0, NG)
    def _(g):
        goff = base + g * GE
        pltpu.sync_copy(epair_hbm.at[pl.ds(goff, GE), pl.ds(0, H)], ebuf)
        cps = []
        for k in range(GRP):
            ck = g * GRP + k
            cps.append(pltpu.async_copy(
                ebuf.at[pl.ds(k * CH, CH)], acc.at[idx_c.at[ck]], sem,
                add=True))
        for cp in cps:
            cp.wait()

    plsc.subcore_barrier()
    pltpu.sync_copy(acc.at[pl.ds(r0, ROWS_PER_TILE)],
                    out_hbm.at[pl.ds(cid * N_NODES + r0, ROWS_PER_TILE)])


def _ln(t, g, b):
    mu = jnp.mean(t, axis=-1, keepdims=True)
    d = t - mu
    var = jnp.mean(d * d, axis=-1, keepdims=True)
    return d * lax.rsqrt(var + 1e-5) * g + b


def _dot(a, b):
    return jnp.dot(a, b, preferred_element_type=_f32)


def _init_body(x, W, b, hout):
    hout[...] = _dot(x[...], W[...]) + b[...]


def _edge_mlp(gcat, ev, W1rc, W1e, b1, g1, be1, W2, b2, g2, be2):
    t = _dot(gcat, W1rc[...]) + _dot(ev, W1e[...]) + b1[...]
    t = jnp.maximum(_ln(t, g1[...], be1[...]), 0.0)
    return _ln(_dot(t, W2[...]) + b2[...], g2[...], be2[...])


def _edge0_body(gcat, ea, edW, edb, W1rc, W1e, b1, g1, be1,
                W2, b2, g2, be2, epair_out):
    ev = _dot(ea[...], edW[...]) + edb[...]
    u = _edge_mlp(gcat[...], ev, W1rc, W1e, b1, g1, be1, W2, b2, g2, be2)
    epair_out[...] = jnp.concatenate([u, ev + u], axis=-1)


def _edge_body(gcat, epair, W1rc, W1e, b1, g1, be1,
               W2, b2, g2, be2, epair_out):
    ev = epair[...][:, H:]
    u = _edge_mlp(gcat[...], ev, W1rc, W1e, b1, g1, be1, W2, b2, g2, be2)
    epair_out[...] = jnp.concatenate([u, ev + u], axis=-1)


def _edge_last_body(gcat, epair, W1rc, W1e, b1, g1, be1,
                    W2, b2, g2, be2, epad_out, enext_out):
    ev = epair[...][:, H:]
    u = _edge_mlp(gcat[...], ev, W1rc, W1e, b1, g1, be1, W2, b2, g2, be2)
    epad_out[...] = jnp.concatenate([u, u], axis=-1)
    enext_out[...] = ev + u


def _node_body(h, parts, W1h, W1a, b1, g1, be1, W2, b2, g2, be2, hout):
    hv = h[...]
    a = parts[0] + parts[1]
    t = _dot(hv, W1h[...]) + _dot(a, W1a[...]) + b1[...]
    t = jnp.maximum(_ln(t, g1[...], be1[...]), 0.0)
    u = _ln(_dot(t, W2[...]) + b2[...], g2[...], be2[...])
    hout[...] = hv + u


BE = 2000
BN = 2000


def _wspec(shape):
    return pl.BlockSpec(shape, lambda i, _s=len(shape): (0,) * _s)


def _tc_init(x, W, b):
    return pl.pallas_call(
        _init_body,
        grid=(N_NODES // BN,),
        in_specs=[pl.BlockSpec((BN, x.shape[1]), lambda i: (i, 0)),
                  _wspec(W.shape), _wspec(b.shape)],
        out_specs=pl.BlockSpec((BN, H), lambda i: (i, 0)),
        out_shape=jax.ShapeDtypeStruct((N_NODES, H), _f32),
    )(x, W, b)


def _tc_edge(body, arrays, weights, out_shapes):
    aspecs = [pl.BlockSpec((BE, a.shape[1]), lambda i: (i, 0)) for a in arrays]
    wspecs = [_wspec(w.shape) for w in weights]
    ospecs = tuple(pl.BlockSpec((BE, s[1]), lambda i: (i, 0))
                   for s in out_shapes)
    oshapes = tuple(jax.ShapeDtypeStruct(s, _f32) for s in out_shapes)
    if len(out_shapes) == 1:
        ospecs, oshapes = ospecs[0], oshapes[0]
    return pl.pallas_call(
        body,
        grid=(N_EDGES // BE,),
        in_specs=aspecs + wspecs,
        out_specs=ospecs,
        out_shape=oshapes,
    )(*arrays, *weights)


def _tc_node(h, parts, weights):
    return pl.pallas_call(
        _node_body,
        grid=(N_NODES // BN,),
        in_specs=[pl.BlockSpec((BN, H), lambda i: (i, 0)),
                  pl.BlockSpec((NC, BN, H), lambda i: (0, i, 0))]
                 + [_wspec(w.shape) for w in weights],
        out_specs=pl.BlockSpec((BN, H), lambda i: (i, 0)),
        out_shape=jax.ShapeDtypeStruct((N_NODES, H), _f32),
    )(h, parts, *weights)


def kernel(x, edge_index, edge_attr, params):
    p = params
    row2 = edge_index[0].reshape(NW, NCH, CH)
    col2 = edge_index[1].reshape(NW, NCH, CH)
    zeros_nodes = jnp.zeros((N_NODES, H), _f32)

    def r1(v):
        return v.reshape(1, H)

    h = _tc_init(x, p['in_W'], r1(p['in_b']))
    e = None
    for l in range(NUM_LAYERS):
        pe = 'l%d_e_' % l
        pn = 'l%d_n_' % l
        W1 = p[pe + 'W1']
        ew = [W1[:H2], W1[H2:], r1(p[pe + 'b1']), r1(p[pe + 'g1']),
              r1(p[pe + 'be1']), p[pe + 'W2'], r1(p[pe + 'b2']),
              r1(p[pe + 'g2']), r1(p[pe + 'be2'])]
        gcat = _sc_gather(h, row2, col2)
        if l == 0:
            epair = _tc_edge(
                _edge0_body, [gcat, edge_attr],
                [p['ed_W'], r1(p['ed_b'])] + ew,
                [(N_EDGES, H2)])
        elif l < NUM_LAYERS - 1:
            epair = _tc_edge(_edge_body, [gcat, epair], ew, [(N_EDGES, H2)])
        else:
            epair, e = _tc_edge(_edge_last_body, [gcat, epair], ew,
                                [(N_EDGES, H2), (N_EDGES, H)])
        parts = _sc_scatter(epair, col2, zeros_nodes).reshape(NC, N_NODES, H)
        nW1 = p[pn + 'W1']
        nw = [nW1[:H], nW1[H:], r1(p[pn + 'b1']), r1(p[pn + 'g1']),
              r1(p[pn + 'be1']), p[pn + 'W2'], r1(p[pn + 'b2']),
              r1(p[pn + 'g2']), r1(p[pn + 'be2'])]
        h = _tc_node(h, parts, nw)
    return (h, e)
